# baseline (speedup 1.0000x reference)
import jax
import jax.numpy as jnp
from jax.experimental import pallas as pl

_EPS = 1e-5
_K = 7


def _pairwise(X):
    N = X.shape[1]
    xx = jnp.sum(X * X, axis=0, keepdims=True)
    ones = jnp.ones((1, N), jnp.float32)
    A = jnp.concatenate([2.0 * X, -ones, -xx], axis=0)
    Bm = jnp.concatenate([X, xx, ones], axis=0)
    return jax.lax.dot_general(
        A, Bm, (((0,), (0,)), ((), ())),
        preferred_element_type=jnp.float32)


def _split(S):
    hi = S.astype(jnp.bfloat16)
    lo = (S - hi.astype(jnp.float32)).astype(jnp.bfloat16)
    return hi, lo


def _edge_block(P, F, S, Wxd, b0, post_w, W1, b1, W2, b2, cout):
    N = P.shape[1]
    pd = _pairwise(P)
    A = jnp.dot(Wxd, F, preferred_element_type=jnp.float32) + b0
    Shi, Slo = _split(S)
    rowid = jax.lax.broadcasted_iota(jnp.int32, (N, N), 0)

    def select(pd):
        m = jnp.max(pd, axis=0, keepdims=True)
        cand = jnp.where(pd == m, rowid, N)
        sel = jnp.min(cand, axis=0, keepdims=True)
        return rowid == sel

    oh0 = select(pd)
    pd = jnp.where(oh0, -jnp.inf, pd)

    def body(_, carry):
        pd, acc = carry
        oh = select(pd)
        pd = jnp.where(oh, -jnp.inf, pd)
        ohb = oh.astype(jnp.bfloat16)
        G = (jnp.dot(Shi, ohb, preferred_element_type=jnp.float32)
             + jnp.dot(Slo, ohb, preferred_element_type=jnp.float32))
        if post_w is not None:
            G = jnp.dot(post_w, G, preferred_element_type=jnp.float32)
        Y = jax.nn.relu(A + G)
        Y = jax.nn.relu(jnp.dot(W1, Y, preferred_element_type=jnp.float32) + b1)
        Y = jax.nn.relu(jnp.dot(W2, Y, preferred_element_type=jnp.float32) + b2)
        return pd, acc + Y

    acc0 = jnp.zeros((cout, N), jnp.float32)
    _, acc = jax.lax.fori_loop(0, _K, body, (pd, acc0))
    return acc * (1.0 / _K)


def _body(pts_ref, fts_ref,
          sfts_ref, bfts_ref,
          e1xd_ref, e1d_ref, e1b0_ref, e1w1_ref, e1b1_ref, e1w2_ref, e1b2_ref,
          e2xd_ref, e2d_ref, e2b0_ref, e2w1_ref, e2b1_ref, e2w2_ref, e2b2_ref,
          wsc_ref, bsc_ref, wfus_ref, bfus_ref,
          wfc0_ref, bfc0_ref, wfc1_ref, bfc1_ref,
          out_ref):
    P0 = pts_ref[0]
    F0 = fts_ref[0]
    mask = (jnp.sum(jnp.abs(F0), axis=0, keepdims=True) != 0.0)
    mask = mask.astype(jnp.float32)
    shift = (1.0 - mask) * 1e9
    F = (F0 * sfts_ref[...] + bfts_ref[...]) * mask

    P1 = P0 * mask + shift
    H1 = jnp.dot(e1d_ref[...], F, preferred_element_type=jnp.float32)
    m1 = _edge_block(P1, F, H1, e1xd_ref[...], e1b0_ref[...], None,
                     e1w1_ref[...], e1b1_ref[...], e1w2_ref[...], e1b2_ref[...],
                     32)
    F1 = jax.nn.relu(F + m1) * mask

    P2 = F1 + shift
    m2 = _edge_block(P2, F1, F1, e2xd_ref[...], e2b0_ref[...], e2d_ref[...],
                     e2w1_ref[...], e2b1_ref[...], e2w2_ref[...], e2b2_ref[...],
                     64)
    sc = jnp.dot(wsc_ref[...], F1, preferred_element_type=jnp.float32) \
        + bsc_ref[...]
    F2 = jax.nn.relu(sc + m2) * mask

    Fc = jnp.concatenate([F1, F2], axis=0)
    Yf = jax.nn.relu(
        jnp.dot(wfus_ref[...], Fc, preferred_element_type=jnp.float32)
        + bfus_ref[...]) * mask
    counts = jnp.maximum(jnp.sum(mask), 1.0)
    pooled = jnp.sum(Yf, axis=1, keepdims=True) / counts
    h = jax.nn.relu(
        jnp.dot(wfc0_ref[...], pooled, preferred_element_type=jnp.float32)
        + bfc0_ref[...])
    out_ref[0] = jnp.dot(wfc1_ref[...], h,
                         preferred_element_type=jnp.float32) + bfc1_ref[...]


def kernel(points, features, bn_fts_g, bn_fts_b,
           ec1_w0, ec1_w1, ec1_w2, ec1_g0, ec1_g1, ec1_g2,
           ec1_b0, ec1_b1, ec1_b2,
           ec2_w0, ec2_w1, ec2_w2, ec2_g0, ec2_g1, ec2_g2,
           ec2_b0, ec2_b1, ec2_b2,
           ec2_sc_w, ec2_sc_g, ec2_sc_b,
           fus_w, fus_g, fus_b, fc0_w, fc0_b, fc1_w, fc1_b):
    B, Cin, N = features.shape
    rs = 1.0 / jnp.sqrt(jnp.float32(1.0 + _EPS))

    def fold(W, g, b):
        return (g * rs)[:, None] * W, b[:, None]

    e1w0, e1b0 = fold(ec1_w0, ec1_g0, ec1_b0)
    e1w1, e1b1 = fold(ec1_w1, ec1_g1, ec1_b1)
    e1w2, e1b2 = fold(ec1_w2, ec1_g2, ec1_b2)
    e2w0, e2b0 = fold(ec2_w0, ec2_g0, ec2_b0)
    e2w1, e2b1 = fold(ec2_w1, ec2_g1, ec2_b1)
    e2w2, e2b2 = fold(ec2_w2, ec2_g2, ec2_b2)
    wsc, bsc = fold(ec2_sc_w, ec2_sc_g, ec2_sc_b)
    wfus, bfus = fold(fus_w, fus_g, fus_b)
    e1xd, e1d = e1w0[:, :Cin] - e1w0[:, Cin:], e1w0[:, Cin:]
    e2xd, e2d = e2w0[:, :Cin] - e2w0[:, Cin:], e2w0[:, Cin:]
    sfts = (bn_fts_g * rs)[:, None]
    bfts = bn_fts_b[:, None]
    wfc0, bfc0 = fc0_w, fc0_b[:, None]
    wfc1, bfc1 = fc1_w, fc1_b[:, None]

    def bspec(shape):
        return pl.BlockSpec(shape, lambda b: (0,) * len(shape))

    ws = [sfts, bfts,
          e1xd, e1d, e1b0, e1w1, e1b1, e1w2, e1b2,
          e2xd, e2d, e2b0, e2w1, e2b1, e2w2, e2b2,
          wsc, bsc, wfus, bfus, wfc0, bfc0, wfc1, bfc1]
    in_specs = [pl.BlockSpec((1, 2, N), lambda b: (b, 0, 0)),
                pl.BlockSpec((1, Cin, N), lambda b: (b, 0, 0))]
    in_specs += [bspec(w.shape) for w in ws]

    out = pl.pallas_call(
        _body,
        grid=(B,),
        in_specs=in_specs,
        out_specs=pl.BlockSpec((1, 10, 1), lambda b: (b, 0, 0)),
        out_shape=jax.ShapeDtypeStruct((B, 10, 1), jnp.float32),
    )(points, features, *ws)
    return out.reshape(B, 10)


# speedup vs baseline: 1.0365x; 1.0365x over previous
import jax
import jax.numpy as jnp
from jax.experimental import pallas as pl

_EPS = 1e-5
_K = 7


def _pairwise(X):
    N = X.shape[1]
    xx = jnp.sum(X * X, axis=0, keepdims=True)
    ones = jnp.ones((1, N), jnp.float32)
    A = jnp.concatenate([2.0 * X, -ones, -xx], axis=0)
    Bm = jnp.concatenate([X, xx, ones], axis=0)
    return jax.lax.dot_general(
        A, Bm, (((0,), (0,)), ((), ())),
        preferred_element_type=jnp.float32)


def _split(S):
    hi = S.astype(jnp.bfloat16)
    lo = (S - hi.astype(jnp.float32)).astype(jnp.bfloat16)
    return hi, lo


def _edge_block(P, F, S, Wxd, b0, post_w, W1, b1, W2, b2, cout):
    N = P.shape[1]
    pd = _pairwise(P)
    A = jnp.dot(Wxd, F, preferred_element_type=jnp.float32) + b0
    Shi, Slo = _split(S)
    rowid = jax.lax.broadcasted_iota(jnp.int32, (N, N), 0).astype(jnp.float32)

    def select(pd):
        m = jnp.max(pd, axis=0, keepdims=True)
        cand = jnp.where(pd == m, rowid, jnp.float32(N))
        sel = jnp.min(cand, axis=0, keepdims=True)
        return cand == sel

    oh0 = select(pd)
    pd = jnp.where(oh0, -jnp.inf, pd)

    def body(_, carry):
        pd, acc = carry
        oh = select(pd)
        pd = jnp.where(oh, -jnp.inf, pd)
        ohb = oh.astype(jnp.bfloat16)
        G = (jnp.dot(Shi, ohb, preferred_element_type=jnp.float32)
             + jnp.dot(Slo, ohb, preferred_element_type=jnp.float32))
        if post_w is not None:
            G = jnp.dot(post_w, G, preferred_element_type=jnp.float32)
        Y = jax.nn.relu(A + G)
        Y = jax.nn.relu(jnp.dot(W1, Y, preferred_element_type=jnp.float32) + b1)
        Y = jax.nn.relu(jnp.dot(W2, Y, preferred_element_type=jnp.float32) + b2)
        return pd, acc + Y

    acc0 = jnp.zeros((cout, N), jnp.float32)
    _, acc = jax.lax.fori_loop(0, _K, body, (pd, acc0))
    return acc * (1.0 / _K)


def _body(pts_ref, fts_ref,
          sfts_ref, bfts_ref,
          e1xd_ref, e1d_ref, e1b0_ref, e1w1_ref, e1b1_ref, e1w2_ref, e1b2_ref,
          e2xd_ref, e2d_ref, e2b0_ref, e2w1_ref, e2b1_ref, e2w2_ref, e2b2_ref,
          wsc_ref, bsc_ref, wfus_ref, bfus_ref,
          wfc0_ref, bfc0_ref, wfc1_ref, bfc1_ref,
          out_ref):
    P0 = pts_ref[0]
    F0 = fts_ref[0]
    mask = (jnp.sum(jnp.abs(F0), axis=0, keepdims=True) != 0.0)
    mask = mask.astype(jnp.float32)
    shift = (1.0 - mask) * 1e9
    F = (F0 * sfts_ref[...] + bfts_ref[...]) * mask

    P1 = P0 * mask + shift
    H1 = jnp.dot(e1d_ref[...], F, preferred_element_type=jnp.float32)
    m1 = _edge_block(P1, F, H1, e1xd_ref[...], e1b0_ref[...], None,
                     e1w1_ref[...], e1b1_ref[...], e1w2_ref[...], e1b2_ref[...],
                     32)
    F1 = jax.nn.relu(F + m1) * mask

    P2 = F1 + shift
    m2 = _edge_block(P2, F1, F1, e2xd_ref[...], e2b0_ref[...], e2d_ref[...],
                     e2w1_ref[...], e2b1_ref[...], e2w2_ref[...], e2b2_ref[...],
                     64)
    sc = jnp.dot(wsc_ref[...], F1, preferred_element_type=jnp.float32) \
        + bsc_ref[...]
    F2 = jax.nn.relu(sc + m2) * mask

    Fc = jnp.concatenate([F1, F2], axis=0)
    Yf = jax.nn.relu(
        jnp.dot(wfus_ref[...], Fc, preferred_element_type=jnp.float32)
        + bfus_ref[...]) * mask
    counts = jnp.maximum(jnp.sum(mask), 1.0)
    pooled = jnp.sum(Yf, axis=1, keepdims=True) / counts
    h = jax.nn.relu(
        jnp.dot(wfc0_ref[...], pooled, preferred_element_type=jnp.float32)
        + bfc0_ref[...])
    out_ref[0] = jnp.dot(wfc1_ref[...], h,
                         preferred_element_type=jnp.float32) + bfc1_ref[...]


def kernel(points, features, bn_fts_g, bn_fts_b,
           ec1_w0, ec1_w1, ec1_w2, ec1_g0, ec1_g1, ec1_g2,
           ec1_b0, ec1_b1, ec1_b2,
           ec2_w0, ec2_w1, ec2_w2, ec2_g0, ec2_g1, ec2_g2,
           ec2_b0, ec2_b1, ec2_b2,
           ec2_sc_w, ec2_sc_g, ec2_sc_b,
           fus_w, fus_g, fus_b, fc0_w, fc0_b, fc1_w, fc1_b):
    B, Cin, N = features.shape
    rs = 1.0 / jnp.sqrt(jnp.float32(1.0 + _EPS))

    def fold(W, g, b):
        return (g * rs)[:, None] * W, b[:, None]

    e1w0, e1b0 = fold(ec1_w0, ec1_g0, ec1_b0)
    e1w1, e1b1 = fold(ec1_w1, ec1_g1, ec1_b1)
    e1w2, e1b2 = fold(ec1_w2, ec1_g2, ec1_b2)
    e2w0, e2b0 = fold(ec2_w0, ec2_g0, ec2_b0)
    e2w1, e2b1 = fold(ec2_w1, ec2_g1, ec2_b1)
    e2w2, e2b2 = fold(ec2_w2, ec2_g2, ec2_b2)
    wsc, bsc = fold(ec2_sc_w, ec2_sc_g, ec2_sc_b)
    wfus, bfus = fold(fus_w, fus_g, fus_b)
    e1xd, e1d = e1w0[:, :Cin] - e1w0[:, Cin:], e1w0[:, Cin:]
    e2xd, e2d = e2w0[:, :Cin] - e2w0[:, Cin:], e2w0[:, Cin:]
    sfts = (bn_fts_g * rs)[:, None]
    bfts = bn_fts_b[:, None]
    wfc0, bfc0 = fc0_w, fc0_b[:, None]
    wfc1, bfc1 = fc1_w, fc1_b[:, None]

    def bspec(shape):
        return pl.BlockSpec(shape, lambda b: (0,) * len(shape))

    ws = [sfts, bfts,
          e1xd, e1d, e1b0, e1w1, e1b1, e1w2, e1b2,
          e2xd, e2d, e2b0, e2w1, e2b1, e2w2, e2b2,
          wsc, bsc, wfus, bfus, wfc0, bfc0, wfc1, bfc1]
    in_specs = [pl.BlockSpec((1, 2, N), lambda b: (b, 0, 0)),
                pl.BlockSpec((1, Cin, N), lambda b: (b, 0, 0))]
    in_specs += [bspec(w.shape) for w in ws]

    out = pl.pallas_call(
        _body,
        grid=(B,),
        in_specs=in_specs,
        out_specs=pl.BlockSpec((1, 10, 1), lambda b: (b, 0, 0)),
        out_shape=jax.ShapeDtypeStruct((B, 10, 1), jnp.float32),
    )(points, features, *ws)
    return out.reshape(B, 10)


# speedup vs baseline: 2.0477x; 1.9755x over previous
import jax
import jax.numpy as jnp
from jax.experimental import pallas as pl

_EPS = 1e-5
_K = 7


def _pairwise(X):
    N = X.shape[1]
    xx = jnp.sum(X * X, axis=0, keepdims=True)
    ones = jnp.ones((1, N), jnp.float32)
    A = jnp.concatenate([2.0 * X, -ones, -xx], axis=0)
    Bm = jnp.concatenate([X, xx, ones], axis=0)
    return jax.lax.dot_general(
        A, Bm, (((0,), (0,)), ((), ())),
        preferred_element_type=jnp.float32)


def _split(S):
    hi = S.astype(jnp.bfloat16)
    lo = (S - hi.astype(jnp.float32)).astype(jnp.bfloat16)
    return hi, lo


def _edge_block(P, F, S, Wxd, b0, post_w, W1, b1, W2, b2, cout):
    N = P.shape[1]
    pd = _pairwise(P)
    A = jnp.dot(Wxd, F, preferred_element_type=jnp.float32) + b0
    Shi, Slo = _split(S)
    irow = jax.lax.broadcasted_iota(jnp.int32, (N, N), 0)
    icol = jax.lax.broadcasted_iota(jnp.int32, (N, N), 1)
    rowid = irow.astype(jnp.float32)

    def select(pd):
        m = jnp.max(pd, axis=0, keepdims=True)
        cand = jnp.where(pd == m, rowid, jnp.float32(N))
        sel = jnp.min(cand, axis=0, keepdims=True)
        return cand == sel

    pd = jnp.where(irow == icol, -jnp.inf, pd)

    acc = jnp.zeros((cout, N), jnp.float32)
    for _ in range(_K):
        oh = select(pd)
        pd = jnp.where(oh, -jnp.inf, pd)
        ohb = oh.astype(jnp.bfloat16)
        G = (jnp.dot(Shi, ohb, preferred_element_type=jnp.float32)
             + jnp.dot(Slo, ohb, preferred_element_type=jnp.float32))
        if post_w is not None:
            G = jnp.dot(post_w, G, preferred_element_type=jnp.float32)
        Y = jax.nn.relu(A + G)
        Y = jax.nn.relu(jnp.dot(W1, Y, preferred_element_type=jnp.float32) + b1)
        Y = jax.nn.relu(jnp.dot(W2, Y, preferred_element_type=jnp.float32) + b2)
        acc = acc + Y
    return acc * (1.0 / _K)


def _body(pts_ref, fts_ref,
          sfts_ref, bfts_ref,
          e1xd_ref, e1d_ref, e1b0_ref, e1w1_ref, e1b1_ref, e1w2_ref, e1b2_ref,
          e2xd_ref, e2d_ref, e2b0_ref, e2w1_ref, e2b1_ref, e2w2_ref, e2b2_ref,
          wsc_ref, bsc_ref, wfus_ref, bfus_ref,
          wfc0_ref, bfc0_ref, wfc1_ref, bfc1_ref,
          out_ref):
    P0 = pts_ref[0]
    F0 = fts_ref[0]
    mask = (jnp.sum(jnp.abs(F0), axis=0, keepdims=True) != 0.0)
    mask = mask.astype(jnp.float32)
    shift = (1.0 - mask) * 1e9
    F = (F0 * sfts_ref[...] + bfts_ref[...]) * mask

    P1 = P0 * mask + shift
    H1 = jnp.dot(e1d_ref[...], F, preferred_element_type=jnp.float32)
    m1 = _edge_block(P1, F, H1, e1xd_ref[...], e1b0_ref[...], None,
                     e1w1_ref[...], e1b1_ref[...], e1w2_ref[...], e1b2_ref[...],
                     32)
    F1 = jax.nn.relu(F + m1) * mask

    P2 = F1 + shift
    m2 = _edge_block(P2, F1, F1, e2xd_ref[...], e2b0_ref[...], e2d_ref[...],
                     e2w1_ref[...], e2b1_ref[...], e2w2_ref[...], e2b2_ref[...],
                     64)
    sc = jnp.dot(wsc_ref[...], F1, preferred_element_type=jnp.float32) \
        + bsc_ref[...]
    F2 = jax.nn.relu(sc + m2) * mask

    Fc = jnp.concatenate([F1, F2], axis=0)
    Yf = jax.nn.relu(
        jnp.dot(wfus_ref[...], Fc, preferred_element_type=jnp.float32)
        + bfus_ref[...]) * mask
    counts = jnp.maximum(jnp.sum(mask), 1.0)
    pooled = jnp.sum(Yf, axis=1, keepdims=True) / counts
    h = jax.nn.relu(
        jnp.dot(wfc0_ref[...], pooled, preferred_element_type=jnp.float32)
        + bfc0_ref[...])
    out_ref[0] = jnp.dot(wfc1_ref[...], h,
                         preferred_element_type=jnp.float32) + bfc1_ref[...]


def kernel(points, features, bn_fts_g, bn_fts_b,
           ec1_w0, ec1_w1, ec1_w2, ec1_g0, ec1_g1, ec1_g2,
           ec1_b0, ec1_b1, ec1_b2,
           ec2_w0, ec2_w1, ec2_w2, ec2_g0, ec2_g1, ec2_g2,
           ec2_b0, ec2_b1, ec2_b2,
           ec2_sc_w, ec2_sc_g, ec2_sc_b,
           fus_w, fus_g, fus_b, fc0_w, fc0_b, fc1_w, fc1_b):
    B, Cin, N = features.shape
    rs = 1.0 / jnp.sqrt(jnp.float32(1.0 + _EPS))

    def fold(W, g, b):
        return (g * rs)[:, None] * W, b[:, None]

    e1w0, e1b0 = fold(ec1_w0, ec1_g0, ec1_b0)
    e1w1, e1b1 = fold(ec1_w1, ec1_g1, ec1_b1)
    e1w2, e1b2 = fold(ec1_w2, ec1_g2, ec1_b2)
    e2w0, e2b0 = fold(ec2_w0, ec2_g0, ec2_b0)
    e2w1, e2b1 = fold(ec2_w1, ec2_g1, ec2_b1)
    e2w2, e2b2 = fold(ec2_w2, ec2_g2, ec2_b2)
    wsc, bsc = fold(ec2_sc_w, ec2_sc_g, ec2_sc_b)
    wfus, bfus = fold(fus_w, fus_g, fus_b)
    e1xd, e1d = e1w0[:, :Cin] - e1w0[:, Cin:], e1w0[:, Cin:]
    e2xd, e2d = e2w0[:, :Cin] - e2w0[:, Cin:], e2w0[:, Cin:]
    sfts = (bn_fts_g * rs)[:, None]
    bfts = bn_fts_b[:, None]
    wfc0, bfc0 = fc0_w, fc0_b[:, None]
    wfc1, bfc1 = fc1_w, fc1_b[:, None]

    def bspec(shape):
        return pl.BlockSpec(shape, lambda b: (0,) * len(shape))

    ws = [sfts, bfts,
          e1xd, e1d, e1b0, e1w1, e1b1, e1w2, e1b2,
          e2xd, e2d, e2b0, e2w1, e2b1, e2w2, e2b2,
          wsc, bsc, wfus, bfus, wfc0, bfc0, wfc1, bfc1]
    in_specs = [pl.BlockSpec((1, 2, N), lambda b: (b, 0, 0)),
                pl.BlockSpec((1, Cin, N), lambda b: (b, 0, 0))]
    in_specs += [bspec(w.shape) for w in ws]

    out = pl.pallas_call(
        _body,
        grid=(B,),
        in_specs=in_specs,
        out_specs=pl.BlockSpec((1, 10, 1), lambda b: (b, 0, 0)),
        out_shape=jax.ShapeDtypeStruct((B, 10, 1), jnp.float32),
    )(points, features, *ws)
    return out.reshape(B, 10)


# speedup vs baseline: 2.3624x; 1.1537x over previous
import jax
import jax.numpy as jnp
from jax.experimental import pallas as pl

_EPS = 1e-5
_K = 7


def _pairwise(X):
    N = X.shape[1]
    xx = jnp.sum(X * X, axis=0, keepdims=True)
    ones = jnp.ones((1, N), jnp.float32)
    A = jnp.concatenate([2.0 * X, -ones, -xx], axis=0)
    Bm = jnp.concatenate([X, xx, ones], axis=0)
    return jax.lax.dot_general(
        A, Bm, (((0,), (0,)), ((), ())),
        preferred_element_type=jnp.float32)


def _edge_block(P, F, S, Wxd, b0, post_w, W1, b1, W2, b2, cout):
    N = P.shape[1]
    pd = _pairwise(P)
    A = jnp.dot(Wxd, F, preferred_element_type=jnp.float32) + b0
    irow = jax.lax.broadcasted_iota(jnp.int32, (N, N), 0)
    icol = jax.lax.broadcasted_iota(jnp.int32, (N, N), 1)
    rowid = irow.astype(jnp.float32)

    def select(pd):
        m = jnp.max(pd, axis=0, keepdims=True)
        cand = jnp.where(pd == m, rowid, jnp.float32(N))
        sel = jnp.min(cand, axis=0, keepdims=True)
        return cand == sel

    pd = jnp.where(irow == icol, -jnp.inf, pd)

    acc = jnp.zeros((cout, N), jnp.float32)
    for _ in range(_K):
        oh = select(pd)
        pd = jnp.where(oh, -jnp.inf, pd)
        G = jnp.dot(S, oh.astype(jnp.float32),
                    preferred_element_type=jnp.float32)
        if post_w is not None:
            G = jnp.dot(post_w, G, preferred_element_type=jnp.float32)
        Y = jax.nn.relu(A + G)
        Y = jax.nn.relu(jnp.dot(W1, Y, preferred_element_type=jnp.float32) + b1)
        Y = jax.nn.relu(jnp.dot(W2, Y, preferred_element_type=jnp.float32) + b2)
        acc = acc + Y
    return acc * (1.0 / _K)


def _body(pts_ref, fts_ref,
          sfts_ref, bfts_ref,
          e1xd_ref, e1d_ref, e1b0_ref, e1w1_ref, e1b1_ref, e1w2_ref, e1b2_ref,
          e2xd_ref, e2d_ref, e2b0_ref, e2w1_ref, e2b1_ref, e2w2_ref, e2b2_ref,
          wsc_ref, bsc_ref, wfus_ref, bfus_ref,
          wfc0_ref, bfc0_ref, wfc1_ref, bfc1_ref,
          out_ref):
    P0 = pts_ref[0]
    F0 = fts_ref[0]
    mask = (jnp.sum(jnp.abs(F0), axis=0, keepdims=True) != 0.0)
    mask = mask.astype(jnp.float32)
    shift = (1.0 - mask) * 1e9
    F = (F0 * sfts_ref[...] + bfts_ref[...]) * mask

    P1 = P0 * mask + shift
    H1 = jnp.dot(e1d_ref[...], F, preferred_element_type=jnp.float32)
    m1 = _edge_block(P1, F, H1, e1xd_ref[...], e1b0_ref[...], None,
                     e1w1_ref[...], e1b1_ref[...], e1w2_ref[...], e1b2_ref[...],
                     32)
    F1 = jax.nn.relu(F + m1) * mask

    P2 = F1 + shift
    m2 = _edge_block(P2, F1, F1, e2xd_ref[...], e2b0_ref[...], e2d_ref[...],
                     e2w1_ref[...], e2b1_ref[...], e2w2_ref[...], e2b2_ref[...],
                     64)
    sc = jnp.dot(wsc_ref[...], F1, preferred_element_type=jnp.float32) \
        + bsc_ref[...]
    F2 = jax.nn.relu(sc + m2) * mask

    Fc = jnp.concatenate([F1, F2], axis=0)
    Yf = jax.nn.relu(
        jnp.dot(wfus_ref[...], Fc, preferred_element_type=jnp.float32)
        + bfus_ref[...]) * mask
    counts = jnp.maximum(jnp.sum(mask), 1.0)
    pooled = jnp.sum(Yf, axis=1, keepdims=True) / counts
    h = jax.nn.relu(
        jnp.dot(wfc0_ref[...], pooled, preferred_element_type=jnp.float32)
        + bfc0_ref[...])
    out_ref[0] = jnp.dot(wfc1_ref[...], h,
                         preferred_element_type=jnp.float32) + bfc1_ref[...]


def kernel(points, features, bn_fts_g, bn_fts_b,
           ec1_w0, ec1_w1, ec1_w2, ec1_g0, ec1_g1, ec1_g2,
           ec1_b0, ec1_b1, ec1_b2,
           ec2_w0, ec2_w1, ec2_w2, ec2_g0, ec2_g1, ec2_g2,
           ec2_b0, ec2_b1, ec2_b2,
           ec2_sc_w, ec2_sc_g, ec2_sc_b,
           fus_w, fus_g, fus_b, fc0_w, fc0_b, fc1_w, fc1_b):
    B, Cin, N = features.shape
    rs = 1.0 / jnp.sqrt(jnp.float32(1.0 + _EPS))

    def fold(W, g, b):
        return (g * rs)[:, None] * W, b[:, None]

    e1w0, e1b0 = fold(ec1_w0, ec1_g0, ec1_b0)
    e1w1, e1b1 = fold(ec1_w1, ec1_g1, ec1_b1)
    e1w2, e1b2 = fold(ec1_w2, ec1_g2, ec1_b2)
    e2w0, e2b0 = fold(ec2_w0, ec2_g0, ec2_b0)
    e2w1, e2b1 = fold(ec2_w1, ec2_g1, ec2_b1)
    e2w2, e2b2 = fold(ec2_w2, ec2_g2, ec2_b2)
    wsc, bsc = fold(ec2_sc_w, ec2_sc_g, ec2_sc_b)
    wfus, bfus = fold(fus_w, fus_g, fus_b)
    e1xd, e1d = e1w0[:, :Cin] - e1w0[:, Cin:], e1w0[:, Cin:]
    e2xd, e2d = e2w0[:, :Cin] - e2w0[:, Cin:], e2w0[:, Cin:]
    sfts = (bn_fts_g * rs)[:, None]
    bfts = bn_fts_b[:, None]
    wfc0, bfc0 = fc0_w, fc0_b[:, None]
    wfc1, bfc1 = fc1_w, fc1_b[:, None]

    def bspec(shape):
        return pl.BlockSpec(shape, lambda b: (0,) * len(shape))

    ws = [sfts, bfts,
          e1xd, e1d, e1b0, e1w1, e1b1, e1w2, e1b2,
          e2xd, e2d, e2b0, e2w1, e2b1, e2w2, e2b2,
          wsc, bsc, wfus, bfus, wfc0, bfc0, wfc1, bfc1]
    in_specs = [pl.BlockSpec((1, 2, N), lambda b: (b, 0, 0)),
                pl.BlockSpec((1, Cin, N), lambda b: (b, 0, 0))]
    in_specs += [bspec(w.shape) for w in ws]

    out = pl.pallas_call(
        _body,
        grid=(B,),
        in_specs=in_specs,
        out_specs=pl.BlockSpec((1, 10, 1), lambda b: (b, 0, 0)),
        out_shape=jax.ShapeDtypeStruct((B, 10, 1), jnp.float32),
    )(points, features, *ws)
    return out.reshape(B, 10)
